# baseline (speedup 1.0000x reference)
import math

import jax
import jax.numpy as jnp
from jax.experimental import pallas as pl
from jax.experimental.pallas import tpu as pltpu

_C = math.sqrt(2.0 / math.pi)
_LOG2E = 1.4426950408889634
_C2 = -2.0 * _C * _LOG2E
_C2A = -2.0 * _C * 0.044715 * _LOG2E


def _fused_body(scal_ref, x_ref, protos_ref, o_ref):
    tau = scal_ref[0]
    gamma = scal_ref[1]
    alpha = scal_ref[2]
    k = protos_ref.shape[0]

    p = protos_ref[...]
    pn = p / jnp.maximum(
        jnp.sqrt(jnp.sum(p * p, axis=1, keepdims=True)), 1e-12)

    x = x_ref[...]
    u = x * (_C2 + _C2A * (x * x))
    out = x / (1.0 + jnp.exp2(u))

    norm = jnp.sqrt(jnp.sum(out * out, axis=1, keepdims=True))
    dots = jax.lax.dot_general(
        out, pn, (((1,), (1,)), ((), ())),
        preferred_element_type=jnp.float32)
    sims = dots / jnp.maximum(norm, 1e-12)

    s = sims * tau
    m = jnp.max(s, axis=1, keepdims=True)
    lse = jnp.log(jnp.sum(jnp.exp(s - m), axis=1, keepdims=True)) + m
    soft_max_sim = lse / tau - math.log(k) / tau

    gate = 1.0 - alpha + alpha * jnp.exp(-gamma * soft_max_sim)
    o_ref[...] = out * gate


def kernel(x, protos, log_tau, log_gamma, log_blend):
    B, T, D = x.shape
    K = protos.shape[0]
    BT = B * T
    xf = x.reshape(BT, D)

    scal = jnp.stack(
        [jnp.exp(log_tau), jnp.exp(log_gamma), jax.nn.sigmoid(log_blend)])

    BR = 256
    while BT % BR:
        BR //= 2
    grid = (BT // BR,)

    out = pl.pallas_call(
        _fused_body,
        grid=grid,
        in_specs=[
            pl.BlockSpec(memory_space=pltpu.SMEM),
            pl.BlockSpec((BR, D), lambda i: (i, 0)),
            pl.BlockSpec((K, D), lambda i: (0, 0)),
        ],
        out_specs=pl.BlockSpec((BR, D), lambda i: (i, 0)),
        out_shape=jax.ShapeDtypeStruct((BT, D), x.dtype),
        compiler_params=pltpu.CompilerParams(
            dimension_semantics=("parallel",),
            vmem_limit_bytes=100 * 1024 * 1024),
    )(scal, xf, protos)
    return out.reshape(B, T, D)


# speedup vs baseline: 1.0876x; 1.0876x over previous
import math

import jax
import jax.numpy as jnp
from jax.experimental import pallas as pl
from jax.experimental.pallas import tpu as pltpu

_C = math.sqrt(2.0 / math.pi)
_LOG2E = 1.4426950408889634
_C2 = -2.0 * _C * _LOG2E
_C2A = -2.0 * _C * 0.044715 * _LOG2E


def _fused_body(scal_ref, x_ref, protos_ref, o_ref):
    tau = scal_ref[0]
    gamma = scal_ref[1]
    alpha = scal_ref[2]
    k = protos_ref.shape[0]

    p = protos_ref[...]
    pn = p / jnp.maximum(
        jnp.sqrt(jnp.sum(p * p, axis=1, keepdims=True)), 1e-12)

    x = x_ref[...]
    u = x * (_C2 + _C2A * (x * x))
    out = x / (1.0 + jnp.exp2(u))

    norm = jnp.sqrt(jnp.sum(out * out, axis=1, keepdims=True))
    dots = jax.lax.dot_general(
        out, pn, (((1,), (1,)), ((), ())),
        preferred_element_type=jnp.float32)
    sims = dots / jnp.maximum(norm, 1e-12)

    s = sims * tau
    m = jnp.max(s, axis=1, keepdims=True)
    lse = jnp.log(jnp.sum(jnp.exp(s - m), axis=1, keepdims=True)) + m
    soft_max_sim = lse / tau - math.log(k) / tau

    gate = 1.0 - alpha + alpha * jnp.exp(-gamma * soft_max_sim)
    o_ref[...] = out * gate


def kernel(x, protos, log_tau, log_gamma, log_blend):
    B, T, D = x.shape
    K = protos.shape[0]
    BT = B * T
    xf = x.reshape(BT, D)

    scal = jnp.stack(
        [jnp.exp(log_tau), jnp.exp(log_gamma), jax.nn.sigmoid(log_blend)])

    BR = 512
    while BT % BR:
        BR //= 2
    grid = (BT // BR,)

    out = pl.pallas_call(
        _fused_body,
        grid=grid,
        in_specs=[
            pl.BlockSpec(memory_space=pltpu.SMEM),
            pl.BlockSpec((BR, D), lambda i: (i, 0)),
            pl.BlockSpec((K, D), lambda i: (0, 0)),
        ],
        out_specs=pl.BlockSpec((BR, D), lambda i: (i, 0)),
        out_shape=jax.ShapeDtypeStruct((BT, D), x.dtype),
        compiler_params=pltpu.CompilerParams(
            dimension_semantics=("parallel",),
            vmem_limit_bytes=100 * 1024 * 1024),
    )(scal, xf, protos)
    return out.reshape(B, T, D)


# speedup vs baseline: 1.0969x; 1.0085x over previous
import math

import jax
import jax.numpy as jnp
from jax.experimental import pallas as pl
from jax.experimental.pallas import tpu as pltpu

_C = math.sqrt(2.0 / math.pi)
_LOG2E = 1.4426950408889634
_C2 = -2.0 * _C * _LOG2E
_C2A = -2.0 * _C * 0.044715 * _LOG2E


def _fused_body(lt_ref, lg_ref, lb_ref, x_ref, protos_ref, o_ref):
    tau = jnp.exp(lt_ref[0])
    gamma = jnp.exp(lg_ref[0])
    alpha = 1.0 / (1.0 + jnp.exp(-lb_ref[0]))
    k = protos_ref.shape[0]

    p = protos_ref[...]
    pn = p / jnp.maximum(
        jnp.sqrt(jnp.sum(p * p, axis=1, keepdims=True)), 1e-12)

    x = x_ref[...]
    u = x * (_C2 + _C2A * (x * x))
    out = x / (1.0 + jnp.exp2(u))

    norm = jnp.sqrt(jnp.sum(out * out, axis=1, keepdims=True))
    dots = jax.lax.dot_general(
        out, pn, (((1,), (1,)), ((), ())),
        preferred_element_type=jnp.float32)
    sims = dots / jnp.maximum(norm, 1e-12)

    s = sims * tau
    m = jnp.max(s, axis=1, keepdims=True)
    lse = jnp.log(jnp.sum(jnp.exp(s - m), axis=1, keepdims=True)) + m
    soft_max_sim = lse / tau - math.log(k) / tau

    gate = 1.0 - alpha + alpha * jnp.exp(-gamma * soft_max_sim)
    o_ref[...] = out * gate


def kernel(x, protos, log_tau, log_gamma, log_blend):
    B, T, D = x.shape
    K = protos.shape[0]
    BT = B * T
    xf = x.reshape(BT, D)

    BR = 512
    while BT % BR:
        BR //= 2
    grid = (BT // BR,)

    out = pl.pallas_call(
        _fused_body,
        grid=grid,
        in_specs=[
            pl.BlockSpec(memory_space=pltpu.SMEM),
            pl.BlockSpec(memory_space=pltpu.SMEM),
            pl.BlockSpec(memory_space=pltpu.SMEM),
            pl.BlockSpec((BR, D), lambda i: (i, 0)),
            pl.BlockSpec((K, D), lambda i: (0, 0)),
        ],
        out_specs=pl.BlockSpec((BR, D), lambda i: (i, 0)),
        out_shape=jax.ShapeDtypeStruct((BT, D), x.dtype),
        compiler_params=pltpu.CompilerParams(
            dimension_semantics=("parallel",),
            vmem_limit_bytes=100 * 1024 * 1024),
    )(log_tau.reshape(1), log_gamma.reshape(1), log_blend.reshape(1),
      xf, protos)
    return out.reshape(B, T, D)
